# Initial kernel scaffold; baseline (speedup 1.0000x reference)
import jax
import jax.numpy as jnp
from jax.experimental import pallas as pl


def kernel(pred, target, codebook):
    raise NotImplementedError("write your pallas kernel here")


# speedup vs baseline: 1.1625x; 1.1625x over previous
import functools

import jax
import jax.numpy as jnp
from jax import lax
from jax.experimental import pallas as pl
from jax.experimental.pallas import tpu as pltpu
from jax.experimental.pallas import tpu_sc as plsc

N = 2048
D = 1024
K = 8192

_info = plsc.get_sparse_core_info()
_NC = _info.num_cores
_NS = _info.num_subcores
_NW = _NC * _NS
_BPW = N // _NW


def _sc_gather(codebook, target):
  mesh = plsc.VectorSubcoreMesh(core_axis_name="c", subcore_axis_name="s")

  @functools.partial(
      pl.kernel,
      mesh=mesh,
      out_type=jax.ShapeDtypeStruct((N, D), jnp.float32),
      scratch_types=[
          pltpu.VMEM((_BPW,), jnp.int32),
          pltpu.VMEM((_BPW, D), jnp.float32),
          pltpu.SemaphoreType.DMA,
      ],
  )
  def k(table_hbm, idx_hbm, out_hbm, idx_v, rows_v, sem):
    wid = lax.axis_index("s") * _NC + lax.axis_index("c")
    base = wid * _BPW
    pltpu.sync_copy(idx_hbm.at[pl.ds(base, _BPW)], idx_v)
    pltpu.async_copy(table_hbm.at[idx_v], rows_v, sem).wait()
    pltpu.sync_copy(rows_v, out_hbm.at[pl.ds(base, _BPW)])

  return k(codebook, target)


_BJ = 256


def _loss_body(pred_ref, e_ref, mask_ref, out_ref, p2_ref):
  j = pl.program_id(0)
  nj = pl.num_programs(0)

  @pl.when(j == 0)
  def _():
    p2_ref[...] = jnp.sum(pred_ref[...] * pred_ref[...], axis=1,
                          keepdims=True)
    out_ref[0, 0] = 0.0

  e = e_ref[...]
  g = lax.dot_general(pred_ref[...], e, (((1,), (1,)), ((), ())),
                      preferred_element_type=jnp.float32)
  t2 = jnp.sum(e * e, axis=1)
  d2 = p2_ref[...] + t2[None, :] - 2.0 * g
  dist = jnp.sqrt(jnp.maximum(d2, 0.0))
  part = jnp.sum(dist * mask_ref[...])
  acc = out_ref[0, 0] + part

  @pl.when(j < nj - 1)
  def _():
    out_ref[0, 0] = acc

  @pl.when(j == nj - 1)
  def _():
    msum = jnp.sum(mask_ref[...])
    out_ref[0, 0] = acc / (msum * D)


def kernel(pred, target, codebook):
  emb = _sc_gather(codebook, target)
  maskf = (target != -1).astype(jnp.float32).reshape(N, 1)

  out = pl.pallas_call(
      _loss_body,
      grid=(N // _BJ,),
      in_specs=[
          pl.BlockSpec((N, D), lambda j: (0, 0)),
          pl.BlockSpec((_BJ, D), lambda j: (j, 0)),
          pl.BlockSpec((N, 1), lambda j: (0, 0)),
      ],
      out_specs=pl.BlockSpec(memory_space=pltpu.SMEM),
      out_shape=jax.ShapeDtypeStruct((1, 1), jnp.float32),
      scratch_shapes=[pltpu.VMEM((N, 1), jnp.float32)],
  )(pred, emb, maskf)
  return out[0, 0]
